# Initial kernel scaffold; baseline (speedup 1.0000x reference)
import jax
import jax.numpy as jnp
from jax.experimental import pallas as pl


def kernel(vertex_attr, edgeij_pair, edge_attr):
    raise NotImplementedError("write your pallas kernel here")


# speedup vs baseline: 12.7651x; 12.7651x over previous
import functools

import jax
import jax.numpy as jnp
from jax import lax
from jax.experimental import pallas as pl
from jax.experimental.pallas import tpu as pltpu
from jax.experimental.pallas import tpu_sc as plsc

_THETA = 0.25
_L = 16
_NC = 2
_NS = 16


@functools.partial(jax.jit, static_argnums=(2,))
def _segmin_edge_update(row, a, n_nodes):
    E = row.shape[0]
    ept1 = E // _NS
    ept2 = E // (_NC * _NS)
    npad = ((n_nodes + _L * _NS - 1) // (_L * _NS)) * (_L * _NS)
    chunk = npad // _NS
    assert ept1 % _L == 0 and ept2 % _L == 0 and ept1 % 8 == 0 and ept2 % 8 == 0

    mesh = plsc.VectorSubcoreMesh(core_axis_name="c", subcore_axis_name="s")

    @functools.partial(
        pl.kernel,
        out_type=jax.ShapeDtypeStruct((E,), jnp.float32),
        mesh=mesh,
        compiler_params=pltpu.CompilerParams(needs_layout_passes=False),
        scratch_types=[
            pltpu.VMEM((ept1,), jnp.int32),
            pltpu.VMEM((ept1,), jnp.float32),
            pltpu.VMEM((npad,), jnp.float32),
            pltpu.VMEM((npad,), jnp.float32),
            pltpu.VMEM_SHARED((_NS, npad), jnp.float32),
            pltpu.VMEM_SHARED((npad,), jnp.float32),
        ],
    )
    def sc_kernel(row_hbm, a_hbm, out_hbm, row_v, a_v, tab_v, red_v, sp_tab, sp_red):
        cid = lax.axis_index("c")
        sid = lax.axis_index("s")

        base1 = sid * ept1
        pltpu.sync_copy(row_hbm.at[pl.ds(base1, ept1)], row_v)
        pltpu.sync_copy(a_hbm.at[pl.ds(base1, ept1)], a_v)

        def init_body(i, c):
            tab_v[pl.ds(i * _L, _L)] = jnp.full((_L,), jnp.inf, jnp.float32)
            return c
        lax.fori_loop(0, npad // _L, init_body, 0)

        def p1_body(i, c):
            off = i * _L
            idx = row_v[pl.ds(off, _L)]
            a16 = a_v[pl.ds(off, _L)]
            cur = plsc.load_gather(tab_v, [idx])
            lost = a16 < cur

            def wcond(lost_m):
                return jnp.any(lost_m)

            def wbody(lost_m):
                plsc.store_scatter(tab_v, [idx], a16, mask=lost_m)
                chk = plsc.load_gather(tab_v, [idx])
                return lost_m & (a16 < chk)

            lax.while_loop(wcond, wbody, lost)
            return c
        lax.fori_loop(0, ept1 // _L, p1_body, 0)

        pltpu.sync_copy(tab_v, sp_tab.at[sid])
        plsc.subcore_barrier()

        cb = sid * chunk
        for r in range(_NS):
            pltpu.sync_copy(sp_tab.at[r, pl.ds(cb, chunk)],
                            red_v.at[pl.ds(r * chunk, chunk)])

        def red_body(j, c):
            jo = j * _L
            m0 = red_v[pl.ds(jo, _L)]

            def racc(r, m):
                return jnp.minimum(m, red_v[pl.ds(r * chunk + jo, _L)])
            m0 = lax.fori_loop(1, _NS, racc, m0)
            tab_v[pl.ds(cb + jo, _L)] = m0
            return c
        lax.fori_loop(0, chunk // _L, red_body, 0)

        pltpu.sync_copy(tab_v.at[pl.ds(cb, chunk)], sp_red.at[pl.ds(cb, chunk)])
        plsc.subcore_barrier()
        pltpu.sync_copy(sp_red, tab_v)

        off2 = cid * ept2

        def p2_body(j, c):
            o = off2 + j * _L
            idx = row_v[pl.ds(o, _L)]
            a16 = a_v[pl.ds(o, _L)]
            m16 = plsc.load_gather(tab_v, [idx])
            a_v[pl.ds(o, _L)] = jnp.maximum(a16 / m16 - _THETA, 0.0)
            return c
        lax.fori_loop(0, ept2 // _L, p2_body, 0)

        pltpu.sync_copy(a_v.at[pl.ds(off2, ept2)],
                        out_hbm.at[pl.ds(base1 + off2, ept2)])

    return sc_kernel(row, a)


def kernel(vertex_attr, edgeij_pair, edge_attr):
    row = edgeij_pair[0]
    a = edge_attr[:, 0]
    return _segmin_edge_update(row, a, vertex_attr.shape[0])


# speedup vs baseline: 17.0212x; 1.3334x over previous
import functools

import jax
import jax.numpy as jnp
from jax import lax
from jax.experimental import pallas as pl
from jax.experimental.pallas import tpu as pltpu
from jax.experimental.pallas import tpu_sc as plsc

_THETA = 0.25
_L = 16
_NC = 2
_NS = 16


@functools.partial(jax.jit, static_argnums=(2,))
def _segmin_edge_update(row, a, n_nodes):
    E = row.shape[0]
    ept1 = E // _NS
    ept2 = E // (_NC * _NS)
    npad = ((n_nodes + _L * _NS - 1) // (_L * _NS)) * (_L * _NS)
    chunk = npad // _NS
    assert ept1 % _L == 0 and ept2 % _L == 0 and ept1 % 8 == 0 and ept2 % 8 == 0

    mesh = plsc.VectorSubcoreMesh(core_axis_name="c", subcore_axis_name="s")

    @functools.partial(
        pl.kernel,
        out_type=jax.ShapeDtypeStruct((E,), jnp.float32),
        mesh=mesh,
        compiler_params=pltpu.CompilerParams(needs_layout_passes=False),
        scratch_types=[
            pltpu.VMEM((ept1,), jnp.int32),
            pltpu.VMEM((ept1,), jnp.float32),
            pltpu.VMEM((npad,), jnp.float32),
            pltpu.VMEM((npad,), jnp.float32),
            pltpu.VMEM_SHARED((_NS, npad), jnp.float32),
            pltpu.VMEM_SHARED((npad,), jnp.float32),
            pltpu.SemaphoreType.DMA,
            pltpu.SemaphoreType.DMA,
        ],
    )
    def sc_kernel(row_hbm, a_hbm, out_hbm, row_v, a_v, tab_v, red_v, sp_tab,
                  sp_red, sem1, sem2):
        cid = lax.axis_index("c")
        sid = lax.axis_index("s")

        base1 = sid * ept1
        cp_row = pltpu.async_copy(row_hbm.at[pl.ds(base1, ept1)], row_v, sem1)
        cp_a = pltpu.async_copy(a_hbm.at[pl.ds(base1, ept1)], a_v, sem2)

        def init_body(i, c):
            tab_v[pl.ds(i * _L, _L)] = jnp.full((_L,), jnp.inf, jnp.float32)
            return c
        lax.fori_loop(0, npad // _L, init_body, 0)
        cp_row.wait()
        cp_a.wait()

        U1 = 5
        trips1 = ept1 // (_L * U1)

        def sweep(_):
            def p1_body(i, acc):
                for u in range(U1):
                    off = (i * U1 + u) * _L
                    idx = row_v[pl.ds(off, _L)]
                    a16 = a_v[pl.ds(off, _L)]
                    cur = plsc.load_gather(tab_v, [idx])
                    lost = a16 < cur
                    plsc.store_scatter(tab_v, [idx], a16, mask=lost)
                    chk = plsc.load_gather(tab_v, [idx])
                    acc = acc | (a16 < chk)
                return acc
            return lax.fori_loop(0, trips1, p1_body,
                                 jnp.zeros((_L,), jnp.bool_))

        fail = sweep(0)
        lax.while_loop(lambda f: jnp.any(f), sweep, fail)

        pltpu.sync_copy(tab_v, sp_tab.at[sid])
        plsc.subcore_barrier()

        cb = sid * chunk
        descs = [pltpu.async_copy(sp_tab.at[r, pl.ds(cb, chunk)],
                                  red_v.at[pl.ds(r * chunk, chunk)], sem1)
                 for r in range(_NS)]
        for d in descs:
            d.wait()

        def red_body(j, c):
            jo = j * _L
            m0 = red_v[pl.ds(jo, _L)]

            def racc(r, m):
                return jnp.minimum(m, red_v[pl.ds(r * chunk + jo, _L)])
            m0 = lax.fori_loop(1, _NS, racc, m0)
            tab_v[pl.ds(cb + jo, _L)] = m0
            return c
        lax.fori_loop(0, chunk // _L, red_body, 0)

        pltpu.sync_copy(tab_v.at[pl.ds(cb, chunk)], sp_red.at[pl.ds(cb, chunk)])
        plsc.subcore_barrier()
        pltpu.sync_copy(sp_red, tab_v)

        off2 = cid * ept2
        U2 = 5
        trips2 = ept2 // (_L * U2)

        def p2_body(j, c):
            for u in range(U2):
                o = off2 + (j * U2 + u) * _L
                idx = row_v[pl.ds(o, _L)]
                a16 = a_v[pl.ds(o, _L)]
                m16 = plsc.load_gather(tab_v, [idx])
                a_v[pl.ds(o, _L)] = jnp.maximum(a16 / m16 - _THETA, 0.0)
            return c
        lax.fori_loop(0, trips2, p2_body, 0)

        pltpu.sync_copy(a_v.at[pl.ds(off2, ept2)],
                        out_hbm.at[pl.ds(base1 + off2, ept2)])

    return sc_kernel(row, a)


def kernel(vertex_attr, edgeij_pair, edge_attr):
    row = edgeij_pair[0]
    a = edge_attr[:, 0]
    return _segmin_edge_update(row, a, vertex_attr.shape[0])
